# baseline (speedup 1.0000x reference)
import functools
import jax
import jax.numpy as jnp
from jax import lax
from jax.experimental import pallas as pl
from jax.experimental.pallas import tpu as pltpu
from jax.experimental.pallas import tpu_sc as plsc

B, S, D = 1024, 200, 128
GB = 64
NC, NS = 2, 16
NW = NC * NS
BW = B // NW


def _copy_body(x_ref, o_ref):
    o_ref[...] = x_ref[...]


def _tc_copy(x):
    return pl.pallas_call(
        _copy_body,
        grid=(B // GB,),
        in_specs=[pl.BlockSpec((GB, S, D), lambda i: (i, 0, 0))],
        out_specs=pl.BlockSpec((GB, S, D), lambda i: (i, 0, 0)),
        out_shape=jax.ShapeDtypeStruct((B, S, D), jnp.float32),
        compiler_params=pltpu.CompilerParams(
            dimension_semantics=("arbitrary",),
        ),
    )(x)


def _sc_body(out_hbm, pos_hbm, mask_hbm, pos_v, idx_v, mask_v, rows_v, sem):
    wid = lax.axis_index("s") * NC + lax.axis_index("c")
    pltpu.sync_copy(pos_hbm.at[pl.ds(wid * BW, BW)], pos_v)
    for j in range(BW // 16):
        batch = wid * BW + j * 16 + lax.iota(jnp.int32, 16)
        idx_v[pl.ds(j * 16, 16)] = pos_v[pl.ds(j * 16, 16)] + batch * S
    pltpu.sync_copy(mask_hbm, mask_v)
    chunks = [mask_v[0, pl.ds(c * 16, 16)] for c in range(D // 16)]
    for r in range(BW):
        for c in range(D // 16):
            rows_v[r, pl.ds(c * 16, 16)] = chunks[c]
    pltpu.async_copy(rows_v, out_hbm.at[idx_v], sem).wait()


_sc_scatter = functools.partial(
    pl.kernel,
    out_type=(),
    mesh=plsc.VectorSubcoreMesh(core_axis_name="c", subcore_axis_name="s"),
    scratch_types=[
        pltpu.VMEM((BW,), jnp.int32),
        pltpu.VMEM((BW,), jnp.int32),
        pltpu.VMEM((1, D), jnp.float32),
        pltpu.VMEM((BW, D), jnp.float32),
        pltpu.SemaphoreType.DMA,
    ],
)(_sc_body)


def kernel(inputs, categories, mask_positions, tokens_embedding):
    del categories
    pos = mask_positions.reshape(B).astype(jnp.int32)
    copied = _tc_copy(inputs)
    out_ref = jax.new_ref(copied.reshape(B * S, D))
    _sc_scatter(out_ref, pos, tokens_embedding)
    return jax.freeze(out_ref).reshape(B, S, D)


# speedup vs baseline: 1.2757x; 1.2757x over previous
import jax
import jax.numpy as jnp
from jax.experimental import pallas as pl
from jax.experimental.pallas import tpu as pltpu

B, S, D = 1024, 200, 128
GB = 128


def _body(pos_ref, x_ref, m_ref, o_ref):
    o_ref[...] = x_ref[...]
    m = m_ref[...]
    for j in range(GB):
        p = pos_ref[j, 0]
        o_ref[j, pl.ds(p, 1), :] = m


def kernel(inputs, categories, mask_positions, tokens_embedding):
    del categories
    pos = mask_positions.astype(jnp.int32)
    grid = (B // GB,)
    out = pl.pallas_call(
        _body,
        grid=grid,
        in_specs=[
            pl.BlockSpec((GB, 1), lambda i: (i, 0), memory_space=pltpu.SMEM),
            pl.BlockSpec((GB, S, D), lambda i: (i, 0, 0)),
            pl.BlockSpec((1, D), lambda i: (0, 0)),
        ],
        out_specs=pl.BlockSpec((GB, S, D), lambda i: (i, 0, 0)),
        out_shape=jax.ShapeDtypeStruct((B, S, D), jnp.float32),
        compiler_params=pltpu.CompilerParams(
            dimension_semantics=("arbitrary",),
        ),
    )(pos, inputs, tokens_embedding)
    return out
